# baseline (speedup 1.0000x reference)
import dataclasses

import jax
import jax.numpy as jnp
from jax import lax
from jax.experimental import pallas as pl
from jax.experimental.pallas import tpu as pltpu
from jax.experimental.pallas import tpu_sc as plsc
from jax.experimental import layout as jex_layout

_NC, _NS = 2, 16
_NW = _NC * _NS
_BLK = 4


def kernel(tensor, table):
    batch, hist = tensor.shape
    table_lin = jex_layout.with_layout_constraint(
        table, jex_layout.Layout(major_to_minor=(0, 1), tiling=())
    )
    idx_flat = tensor.reshape(batch * hist)
    halves = [
        _make_gather(b0, batch // 2, hist, table.shape, table.dtype)(
            table_lin, idx_flat
        )
        for b0 in (0, batch // 2)
    ]
    return jnp.concatenate(halves, axis=0)


def _make_gather(batch0, nbatches, hist, table_shape, dtype):
    vocab, embed = table_shape
    n = nbatches * hist
    row0 = batch0 * hist
    per_wb = nbatches // _NW
    per_w = per_wb * hist
    rows_k = _BLK * hist
    nchunk = per_wb // _BLK

    mesh = plsc.VectorSubcoreMesh(
        core_axis_name="core", subcore_axis_name="subcore"
    )
    cp = pltpu.CompilerParams()
    if "needs_layout_passes" in pltpu.CompilerParams.__dataclass_fields__:
        cp = dataclasses.replace(cp, needs_layout_passes=False)

    @pl.kernel(
        compiler_params=cp,
        out_type=jax.ShapeDtypeStruct((nbatches, hist, embed), dtype),
        mesh=mesh,
        scratch_types=[
            pltpu.VMEM((rows_k,), jnp.int32),
            pltpu.VMEM((rows_k,), jnp.int32),
            pltpu.VMEM((2 * rows_k,), jnp.int32),
            pltpu.VMEM((2 * rows_k,), jnp.int32),
            pltpu.VMEM((2 * rows_k, embed), jnp.float32),
            pltpu.VMEM((2 * rows_k, embed), jnp.float32),
            pltpu.SemaphoreType.DMA,
            pltpu.SemaphoreType.DMA,
            pltpu.SemaphoreType.DMA,
            pltpu.SemaphoreType.DMA,
            pltpu.SemaphoreType.DMA,
            pltpu.SemaphoreType.DMA,
        ],
    )
    def gather_kernel(
        table_hbm, idx_hbm, out_hbm,
        idxk0, idxk1, idxc0, idxc1, dst0, dst1,
        isem0, isem1, gsem0, gsem1, osem0, osem1,
    ):
        wid = lax.axis_index("subcore") * _NC + lax.axis_index("core")
        base = row0 + wid * per_w
        bbase = wid * per_wb
        half_lanes = lax.shift_right_logical(lax.iota(jnp.int32, 16), 1)

        def start_ik(chunk, idxk, isem):
            chunk = jnp.minimum(chunk, nchunk - 1)
            return pltpu.async_copy(
                idx_hbm.at[pl.ds(base + chunk * rows_k, rows_k)], idxk, isem
            )

        def fill(idxk, idxc):
            @pl.loop(0, 2 * rows_k, step=16)
            def _(s):
                v = plsc.load_gather(
                    idxk, [lax.shift_right_logical(s, 1) + half_lanes]
                )
                idxc[pl.ds(s, 16)] = v + v

        def start_gather(idxc, dst, gsem):
            return pltpu.async_copy(table_hbm.at[idxc], dst, gsem)

        def start_out(dst, chunk, osem):
            return pltpu.async_copy(
                dst.at[pl.ds(0, rows_k)].reshape(_BLK, hist, embed),
                out_hbm.at[pl.ds(bbase + chunk * _BLK, _BLK)],
                osem,
            )

        i0 = start_ik(0, idxk0, isem0)
        i1 = start_ik(1, idxk1, isem1)
        i0.wait()
        fill(idxk0, idxc0)
        start_ik(2, idxk0, isem0)
        g0 = start_gather(idxc0, dst0, gsem0)
        i1.wait()
        fill(idxk1, idxc1)
        start_ik(3, idxk1, isem1)
        g1 = start_gather(idxc1, dst1, gsem1)

        @pl.loop(0, (nchunk - 2) // 2)
        def _(k):
            g0.wait()
            o0 = start_out(dst0, 2 * k, osem0)
            i0.wait()
            fill(idxk0, idxc0)
            start_ik(2 * k + 4, idxk0, isem0)
            o0.wait()
            start_gather(idxc0, dst0, gsem0)
            g1.wait()
            o1 = start_out(dst1, 2 * k + 1, osem1)
            i1.wait()
            fill(idxk1, idxc1)
            start_ik(2 * k + 5, idxk1, isem1)
            o1.wait()
            start_gather(idxc1, dst1, gsem1)

        g0.wait()
        o0 = start_out(dst0, nchunk - 2, osem0)
        g1.wait()
        o1 = start_out(dst1, nchunk - 1, osem1)
        i0.wait()
        i1.wait()
        o0.wait()
        o1.wait()

    return gather_kernel


# speedup vs baseline: 1.0992x; 1.0992x over previous
import dataclasses

import jax
import jax.numpy as jnp
from jax import lax
from jax.experimental import pallas as pl
from jax.experimental.pallas import tpu as pltpu
from jax.experimental.pallas import tpu_sc as plsc
from jax.experimental import layout as jex_layout

_NC, _NS = 2, 16
_NW = _NC * _NS
_BLK = 4


def kernel(tensor, table):
    batch, hist = tensor.shape
    vocab, embed = table.shape
    n = batch * hist
    per_wb = batch // _NW
    per_w = per_wb * hist
    rows_k = _BLK * hist
    nchunk = per_wb // _BLK
    idx_flat = tensor.reshape(n)
    table_lin = jex_layout.with_layout_constraint(
        table, jex_layout.Layout(major_to_minor=(0, 1), tiling=())
    )

    mesh = plsc.VectorSubcoreMesh(
        core_axis_name="core", subcore_axis_name="subcore"
    )
    cp = pltpu.CompilerParams()
    if "needs_layout_passes" in pltpu.CompilerParams.__dataclass_fields__:
        cp = dataclasses.replace(cp, needs_layout_passes=False)

    @pl.kernel(
        compiler_params=cp,
        out_type=jax.ShapeDtypeStruct((batch, hist, embed), table.dtype),
        mesh=mesh,
        scratch_types=[
            pltpu.VMEM((rows_k,), jnp.int32),
            pltpu.VMEM((rows_k,), jnp.int32),
            pltpu.VMEM((2 * rows_k,), jnp.int32),
            pltpu.VMEM((2 * rows_k,), jnp.int32),
            pltpu.VMEM((2 * rows_k, embed), jnp.float32),
            pltpu.VMEM((2 * rows_k, embed), jnp.float32),
            pltpu.SemaphoreType.DMA,
            pltpu.SemaphoreType.DMA,
            pltpu.SemaphoreType.DMA,
            pltpu.SemaphoreType.DMA,
            pltpu.SemaphoreType.DMA,
            pltpu.SemaphoreType.DMA,
        ],
    )
    def gather_kernel(
        table_hbm, idx_hbm, out_hbm,
        idxk0, idxk1, idxc0, idxc1, dst0, dst1,
        isem0, isem1, gsem0, gsem1, osem0, osem1,
    ):
        wid = lax.axis_index("subcore") * _NC + lax.axis_index("core")
        base = wid * per_w
        bbase = wid * per_wb
        half_lanes = lax.shift_right_logical(lax.iota(jnp.int32, 16), 1)

        def start_ik(chunk, idxk, isem):
            chunk = jnp.minimum(chunk, nchunk - 1)
            return pltpu.async_copy(
                idx_hbm.at[pl.ds(base + chunk * rows_k, rows_k)], idxk, isem
            )

        def fill(idxk, idxc):
            @pl.loop(0, 2 * rows_k, step=16)
            def _(s):
                v = plsc.load_gather(
                    idxk, [lax.shift_right_logical(s, 1) + half_lanes]
                )
                idxc[pl.ds(s, 16)] = v + v

        def start_gather(idxc, dst, gsem):
            return pltpu.async_copy(table_hbm.at[idxc], dst, gsem)

        def start_out(dst, chunk, osem):
            return pltpu.async_copy(
                dst.at[pl.ds(0, rows_k)].reshape(_BLK, hist, embed),
                out_hbm.at[pl.ds(bbase + chunk * _BLK, _BLK)],
                osem,
            )

        i0 = start_ik(0, idxk0, isem0)
        i1 = start_ik(1, idxk1, isem1)
        i0.wait()
        fill(idxk0, idxc0)
        start_ik(2, idxk0, isem0)
        g0 = start_gather(idxc0, dst0, gsem0)
        i1.wait()
        fill(idxk1, idxc1)
        start_ik(3, idxk1, isem1)
        g1 = start_gather(idxc1, dst1, gsem1)

        @pl.loop(0, (nchunk - 2) // 2)
        def _(k):
            g0.wait()
            o0 = start_out(dst0, 2 * k, osem0)
            i0.wait()
            fill(idxk0, idxc0)
            start_ik(2 * k + 4, idxk0, isem0)
            o0.wait()
            start_gather(idxc0, dst0, gsem0)
            g1.wait()
            o1 = start_out(dst1, 2 * k + 1, osem1)
            i1.wait()
            fill(idxk1, idxc1)
            start_ik(2 * k + 5, idxk1, isem1)
            o1.wait()
            start_gather(idxc1, dst1, gsem1)

        g0.wait()
        o0 = start_out(dst0, nchunk - 2, osem0)
        g1.wait()
        o1 = start_out(dst1, nchunk - 1, osem1)
        i0.wait()
        i1.wait()
        o0.wait()
        o1.wait()

    return gather_kernel(table_lin, idx_flat)
